# baseline (speedup 1.0000x reference)
import functools

import jax
import jax.numpy as jnp
from jax import lax
from jax.experimental import pallas as pl
from jax.experimental.pallas import tpu as pltpu
from jax.experimental.pallas import tpu_sc as plsc

B, N, D = 4, 2048, 1024
BN = 2048
NBLK = N // BN
NSUB = 8
BSUB = BN // NSUB
EPS = 1e-8


def _tc_body(a_ref, b_ref, idx_ref, s_ref):
    b = pl.program_id(0)

    full = b_ref[0]
    ss = jnp.sum(full * full, axis=1)
    r2 = (1.0 / jnp.maximum(jnp.sqrt(ss), EPS)).reshape(1, N)

    a = a_ref[0]
    for j in range(NSUB):
        aj = a[j * BSUB:(j + 1) * BSUB, :]
        dots = lax.dot_general(
            aj, b_ref[0],
            dimension_numbers=(((1,), (1,)), ((), ())),
            preferred_element_type=jnp.float32,
        )
        sd = dots * r2
        mx = jnp.max(sd, axis=1, keepdims=True)
        col = lax.broadcasted_iota(jnp.int32, (BSUB, N), 1)
        cand = jnp.where(sd == mx, col, N)
        amin = jnp.min(cand, axis=1)
        idx_ref[0, 0, pl.ds(j * BSUB, BSUB)] = amin + b * N

    s_ref[0] = (a + b_ref[0]) * 0.5


def _tc_stage(clip1, clip2):
    return pl.pallas_call(
        _tc_body,
        grid=(B,),
        in_specs=[
            pl.BlockSpec((1, BN, D), lambda b: (b, 0, 0)),
            pl.BlockSpec((1, N, D), lambda b: (b, 0, 0)),
        ],
        out_specs=[
            pl.BlockSpec((1, 1, N), lambda b: (b, 0, 0)),
            pl.BlockSpec((1, BN, D), lambda b: (b, 0, 0)),
        ],
        out_shape=[
            jax.ShapeDtypeStruct((B, 1, N), jnp.int32),
            jax.ShapeDtypeStruct((B, N, D), jnp.float32),
        ],
    )(clip1, clip2)


_NC, _NS = 2, 16
NW = _NC * _NS
ROWS = B * N
RPW = ROWS // NW
CHUNK = 64
NCHUNK = RPW // CHUNK

@functools.lru_cache(maxsize=1)
def _make_sc_gather():
    mesh = plsc.VectorSubcoreMesh(
        core_axis_name="c", subcore_axis_name="s",
        num_cores=_NC, num_subcores=_NS)

    @functools.partial(
        pl.kernel,
        mesh=mesh,
        out_type=jax.ShapeDtypeStruct((ROWS, D), jnp.float32),
        scratch_types=[
            pltpu.VMEM((NCHUNK, CHUNK), jnp.int32),
            pltpu.VMEM((CHUNK, D), jnp.float32),
            pltpu.SemaphoreType.DMA,
        ],
    )
    def _sc_gather(s_hbm, gidx_hbm, out_hbm, idx_v, rows_v, sem):
        wid = lax.axis_index("s") * _NC + lax.axis_index("c")
        base = wid * RPW
        pltpu.sync_copy(gidx_hbm.at[wid], idx_v)

        def chunk(c, carry):
            pltpu.async_copy(s_hbm.at[idx_v.at[c]], rows_v, sem).wait()
            pltpu.sync_copy(
                rows_v, out_hbm.at[pl.ds(base + c * CHUNK, CHUNK)])
            return carry

        lax.fori_loop(0, NCHUNK, chunk, 0)

    return _sc_gather


def kernel(clip1_embeddings, clip2_embeddings):
    gidx, s = _tc_stage(clip1_embeddings, clip2_embeddings)
    merged = _make_sc_gather()(
        s.reshape(ROWS, D),
        gidx.reshape(NW, NCHUNK, CHUNK),
    )
    return merged.reshape(B, N, D)


# speedup vs baseline: 1.0632x; 1.0632x over previous
import functools

import jax
import jax.numpy as jnp
from jax import lax
from jax.experimental import pallas as pl
from jax.experimental.pallas import tpu as pltpu
from jax.experimental.pallas import tpu_sc as plsc

B, N, D = 4, 2048, 1024
BN = 2048
NBLK = N // BN
NSUB = 4
BSUB = BN // NSUB
EPS = 1e-8


def _tc_body(a_ref, b_ref, idx_ref, s_ref):
    b = pl.program_id(0)

    full = b_ref[0]
    ss = jnp.sum(full * full, axis=1)
    r2 = (1.0 / jnp.maximum(jnp.sqrt(ss), EPS)).reshape(1, N)

    a = a_ref[0]
    for j in range(NSUB):
        aj = a[j * BSUB:(j + 1) * BSUB, :]
        dots = lax.dot_general(
            aj, b_ref[0],
            dimension_numbers=(((1,), (1,)), ((), ())),
            preferred_element_type=jnp.float32,
        )
        sd = dots * r2
        mx = jnp.max(sd, axis=1, keepdims=True)
        col = lax.broadcasted_iota(jnp.int32, (BSUB, N), 1)
        cand = jnp.where(sd == mx, col, N)
        amin = jnp.min(cand, axis=1)
        idx_ref[0, 0, pl.ds(j * BSUB, BSUB)] = amin + b * N

    s_ref[0] = (a + b_ref[0]) * 0.5


def _tc_stage(clip1, clip2):
    return pl.pallas_call(
        _tc_body,
        grid=(B,),
        in_specs=[
            pl.BlockSpec((1, BN, D), lambda b: (b, 0, 0)),
            pl.BlockSpec((1, N, D), lambda b: (b, 0, 0)),
        ],
        out_specs=[
            pl.BlockSpec((1, 1, N), lambda b: (b, 0, 0)),
            pl.BlockSpec((1, BN, D), lambda b: (b, 0, 0)),
        ],
        out_shape=[
            jax.ShapeDtypeStruct((B, 1, N), jnp.int32),
            jax.ShapeDtypeStruct((B, N, D), jnp.float32),
        ],
    )(clip1, clip2)


_NC, _NS = 2, 16
NW = _NC * _NS
ROWS = B * N
RPW = ROWS // NW
CHUNK = 64
NCHUNK = RPW // CHUNK

@functools.lru_cache(maxsize=1)
def _make_sc_gather():
    mesh = plsc.VectorSubcoreMesh(
        core_axis_name="c", subcore_axis_name="s",
        num_cores=_NC, num_subcores=_NS)

    @functools.partial(
        pl.kernel,
        mesh=mesh,
        out_type=jax.ShapeDtypeStruct((ROWS, D), jnp.float32),
        scratch_types=[
            pltpu.VMEM((NCHUNK, CHUNK), jnp.int32),
            pltpu.VMEM((CHUNK, D), jnp.float32),
            pltpu.SemaphoreType.DMA,
        ],
    )
    def _sc_gather(s_hbm, gidx_hbm, out_hbm, idx_v, rows_v, sem):
        wid = lax.axis_index("s") * _NC + lax.axis_index("c")
        base = wid * RPW
        pltpu.sync_copy(gidx_hbm.at[wid], idx_v)

        def chunk(c, carry):
            pltpu.async_copy(s_hbm.at[idx_v.at[c]], rows_v, sem).wait()
            pltpu.sync_copy(
                rows_v, out_hbm.at[pl.ds(base + c * CHUNK, CHUNK)])
            return carry

        lax.fori_loop(0, NCHUNK, chunk, 0)

    return _sc_gather


def kernel(clip1_embeddings, clip2_embeddings):
    gidx, s = _tc_stage(clip1_embeddings, clip2_embeddings)
    merged = _make_sc_gather()(
        s.reshape(ROWS, D),
        gidx.reshape(NW, NCHUNK, CHUNK),
    )
    return merged.reshape(B, N, D)


# speedup vs baseline: 1.0659x; 1.0026x over previous
import functools

import jax
import jax.numpy as jnp
from jax import lax
from jax.experimental import pallas as pl
from jax.experimental.pallas import tpu as pltpu
from jax.experimental.pallas import tpu_sc as plsc

B, N, D = 4, 2048, 1024
BN = 2048
NBLK = N // BN
NSUB = 4
BSUB = BN // NSUB
EPS = 1e-8


def _tc_body(a_ref, b_ref, idx_ref, s_ref):
    b = pl.program_id(0)

    full = b_ref[0]
    ss = jnp.sum(full * full, axis=1)
    r2 = (1.0 / jnp.maximum(jnp.sqrt(ss), EPS)).reshape(1, N)

    a = a_ref[0]
    b16 = full.astype(jnp.bfloat16)
    for j in range(NSUB):
        aj = a[j * BSUB:(j + 1) * BSUB, :]
        dots = lax.dot_general(
            aj, b16,
            dimension_numbers=(((1,), (1,)), ((), ())),
            preferred_element_type=jnp.float32,
        )
        sd = dots * r2
        mx = jnp.max(sd, axis=1, keepdims=True)
        col = lax.broadcasted_iota(jnp.int32, (BSUB, N), 1)
        cand = jnp.where(sd == mx, col, N)
        amin = jnp.min(cand, axis=1)
        idx_ref[0, 0, pl.ds(j * BSUB, BSUB)] = amin + b * N

    s_ref[0] = (a + b_ref[0]) * 0.5


def _tc_stage(clip1, clip2):
    return pl.pallas_call(
        _tc_body,
        grid=(B,),
        in_specs=[
            pl.BlockSpec((1, BN, D), lambda b: (b, 0, 0)),
            pl.BlockSpec((1, N, D), lambda b: (b, 0, 0)),
        ],
        out_specs=[
            pl.BlockSpec((1, 1, N), lambda b: (b, 0, 0)),
            pl.BlockSpec((1, BN, D), lambda b: (b, 0, 0)),
        ],
        out_shape=[
            jax.ShapeDtypeStruct((B, 1, N), jnp.int32),
            jax.ShapeDtypeStruct((B, N, D), jnp.float32),
        ],
    )(clip1, clip2)


_NC, _NS = 2, 16
NW = _NC * _NS
ROWS = B * N
RPW = ROWS // NW
CHUNK = 64
NCHUNK = RPW // CHUNK

@functools.lru_cache(maxsize=1)
def _make_sc_gather():
    mesh = plsc.VectorSubcoreMesh(
        core_axis_name="c", subcore_axis_name="s",
        num_cores=_NC, num_subcores=_NS)

    @functools.partial(
        pl.kernel,
        mesh=mesh,
        out_type=jax.ShapeDtypeStruct((ROWS, D), jnp.float32),
        scratch_types=[
            pltpu.VMEM((NCHUNK, CHUNK), jnp.int32),
            pltpu.VMEM((CHUNK, D), jnp.float32),
            pltpu.SemaphoreType.DMA,
        ],
    )
    def _sc_gather(s_hbm, gidx_hbm, out_hbm, idx_v, rows_v, sem):
        wid = lax.axis_index("s") * _NC + lax.axis_index("c")
        base = wid * RPW
        pltpu.sync_copy(gidx_hbm.at[wid], idx_v)

        def chunk(c, carry):
            pltpu.async_copy(s_hbm.at[idx_v.at[c]], rows_v, sem).wait()
            pltpu.sync_copy(
                rows_v, out_hbm.at[pl.ds(base + c * CHUNK, CHUNK)])
            return carry

        lax.fori_loop(0, NCHUNK, chunk, 0)

    return _sc_gather


def kernel(clip1_embeddings, clip2_embeddings):
    gidx, s = _tc_stage(clip1_embeddings, clip2_embeddings)
    merged = _make_sc_gather()(
        s.reshape(ROWS, D),
        gidx.reshape(NW, NCHUNK, CHUNK),
    )
    return merged.reshape(B, N, D)
